# Initial kernel scaffold; baseline (speedup 1.0000x reference)
import jax
import jax.numpy as jnp
from jax.experimental import pallas as pl


def kernel(x_start, target, W1, b1, g1, beta1, rm1, rv1, W2, b2, g2, beta2, rm2, rv2, W3, b3):
    raise NotImplementedError("write your pallas kernel here")


# speedup vs baseline: 1.2176x; 1.2176x over previous
import functools

import jax
import jax.numpy as jnp
from jax.experimental import pallas as pl
from jax.experimental.pallas import tpu as pltpu

_B, _D, _H, _L = 16384, 32, 64, 100
_LP = 128
_BM = 2048
_GRID = _B // _BM


def _fused_kernel(x_ref, t_ref, w1_ref, c1_ref, w2_ref, c2_ref, w3_ref,
                  c3_ref, out_ref):
    i = pl.program_id(0)

    x = x_ref[...]
    h = jnp.maximum(
        jnp.dot(x, w1_ref[...], preferred_element_type=jnp.float32)
        + c1_ref[...], 0.0)
    h = jnp.maximum(
        jnp.dot(h, w2_ref[...], preferred_element_type=jnp.float32)
        + c2_ref[...], 0.0)
    logits = (jnp.dot(h, w3_ref[...], preferred_element_type=jnp.float32)
              + c3_ref[...])

    m = jnp.max(logits, axis=1, keepdims=True)
    lse = m + jnp.log(jnp.sum(jnp.exp(logits - m), axis=1, keepdims=True))

    lane = jax.lax.broadcasted_iota(jnp.int32, (_BM, _LP), 1)
    onehot = lane == t_ref[...]
    part = (jnp.sum(jnp.where(onehot, logits, 0.0)) - jnp.sum(lse))

    @pl.when(i == 0)
    def _():
        out_ref[0, 0] = 0.0

    out_ref[0, 0] += part

    @pl.when(i == _GRID - 1)
    def _():
        out_ref[0, 0] = out_ref[0, 0] * (-1.0 / _B)


@functools.partial(jax.jit, static_argnames=())
def kernel(x_start, target, W1, b1, g1, beta1, rm1, rv1, W2, b2, g2, beta2,
           rm2, rv2, W3, b3):
    eps = 1e-5
    s1 = g1 / jnp.sqrt(rv1 + eps)
    s2 = g2 / jnp.sqrt(rv2 + eps)
    w1 = W1 * s1[None, :]
    c1 = ((b1 - rm1) * s1 + beta1)[None, :]
    w2 = W2 * s2[None, :]
    c2 = ((b2 - rm2) * s2 + beta2)[None, :]
    w3 = jnp.pad(W3, ((0, 0), (0, _LP - _L)))
    c3 = jnp.pad(b3, (0, _LP - _L), constant_values=-1e30)[None, :]
    tgt = target.astype(jnp.int32).reshape(_B, 1)

    loss = pl.pallas_call(
        _fused_kernel,
        grid=(_GRID,),
        in_specs=[
            pl.BlockSpec((_BM, _D), lambda i: (i, 0)),
            pl.BlockSpec((_BM, 1), lambda i: (i, 0)),
            pl.BlockSpec((_D, _H), lambda i: (0, 0)),
            pl.BlockSpec((1, _H), lambda i: (0, 0)),
            pl.BlockSpec((_H, _H), lambda i: (0, 0)),
            pl.BlockSpec((1, _H), lambda i: (0, 0)),
            pl.BlockSpec((_H, _LP), lambda i: (0, 0)),
            pl.BlockSpec((1, _LP), lambda i: (0, 0)),
        ],
        out_specs=pl.BlockSpec(memory_space=pltpu.SMEM),
        out_shape=jax.ShapeDtypeStruct((1, 1), jnp.float32),
    )(x_start, tgt, w1, c1, w2, c2, w3, c3)
    return loss[0, 0]


# speedup vs baseline: 1.5308x; 1.2572x over previous
import jax
import jax.numpy as jnp
from jax.experimental import pallas as pl
from jax.experimental.pallas import tpu as pltpu

_B, _D, _H, _L = 16384, 32, 64, 100
_BM = 2048
_GRID = _B // _BM


def _fused_kernel(x_ref, t_ref, w1_ref, b1_ref, g1_ref, beta1_ref, rm1_ref,
                  rv1_ref, w2_ref, b2_ref, g2_ref, beta2_ref, rm2_ref,
                  rv2_ref, w3_ref, b3_ref, out_ref):
    i = pl.program_id(0)
    eps = 1e-5

    s1 = g1_ref[...] * jax.lax.rsqrt(rv1_ref[...] + eps)
    c1 = (b1_ref[...] - rm1_ref[...]) * s1 + beta1_ref[...]
    s2 = g2_ref[...] * jax.lax.rsqrt(rv2_ref[...] + eps)
    c2 = (b2_ref[...] - rm2_ref[...]) * s2 + beta2_ref[...]

    x = x_ref[...]
    h = jnp.maximum(
        jnp.dot(x, w1_ref[...] * s1, preferred_element_type=jnp.float32)
        + c1, 0.0)
    h = jnp.maximum(
        jnp.dot(h, w2_ref[...] * s2, preferred_element_type=jnp.float32)
        + c2, 0.0)
    logits = (jnp.dot(h, w3_ref[...], preferred_element_type=jnp.float32)
              + b3_ref[...])

    m = jnp.max(logits, axis=1, keepdims=True)
    lse = m + jnp.log(jnp.sum(jnp.exp(logits - m), axis=1, keepdims=True))

    lane = jax.lax.broadcasted_iota(jnp.int32, (_BM, _L), 1)
    onehot = lane == t_ref[...]
    part = (jnp.sum(jnp.where(onehot, logits, 0.0)) - jnp.sum(lse))

    @pl.when(i == 0)
    def _():
        out_ref[0, 0] = 0.0

    out_ref[0, 0] += part

    @pl.when(i == _GRID - 1)
    def _():
        out_ref[0, 0] = out_ref[0, 0] * (-1.0 / _B)


def kernel(x_start, target, W1, b1, g1, beta1, rm1, rv1, W2, b2, g2, beta2,
           rm2, rv2, W3, b3):
    tgt = target.astype(jnp.int32).reshape(_B, 1)
    row = lambda v: v.reshape(1, -1)

    full = lambda shape: pl.BlockSpec(shape, lambda i: tuple(0 for _ in shape))
    loss = pl.pallas_call(
        _fused_kernel,
        grid=(_GRID,),
        in_specs=[
            pl.BlockSpec((_BM, _D), lambda i: (i, 0)),
            pl.BlockSpec((_BM, 1), lambda i: (i, 0)),
            full((_D, _H)),
            full((1, _H)), full((1, _H)), full((1, _H)),
            full((1, _H)), full((1, _H)),
            full((_H, _H)),
            full((1, _H)), full((1, _H)), full((1, _H)),
            full((1, _H)), full((1, _H)),
            full((_H, _L)),
            full((1, _L)),
        ],
        out_specs=pl.BlockSpec(memory_space=pltpu.SMEM),
        out_shape=jax.ShapeDtypeStruct((1, 1), jnp.float32),
    )(x_start, tgt, W1, row(b1), row(g1), row(beta1), row(rm1), row(rv1),
      W2, row(b2), row(g2), row(beta2), row(rm2), row(rv2), W3, row(b3))
    return loss[0, 0]


# speedup vs baseline: 1.6024x; 1.0468x over previous
import jax
import jax.numpy as jnp
from jax.experimental import pallas as pl
from jax.experimental.pallas import tpu as pltpu

_B, _D, _H, _L = 16384, 32, 64, 100
_BM = 4096
_GRID = _B // _BM


def _fused_kernel(x_ref, t_ref, w1_ref, b1_ref, g1_ref, beta1_ref, rm1_ref,
                  rv1_ref, w2_ref, b2_ref, g2_ref, beta2_ref, rm2_ref,
                  rv2_ref, w3_ref, b3_ref, out_ref):
    i = pl.program_id(0)
    eps = 1e-5

    s1 = g1_ref[...] * jax.lax.rsqrt(rv1_ref[...] + eps)
    c1 = (b1_ref[...] - rm1_ref[...]) * s1 + beta1_ref[...]
    s2 = g2_ref[...] * jax.lax.rsqrt(rv2_ref[...] + eps)
    c2 = (b2_ref[...] - rm2_ref[...]) * s2 + beta2_ref[...]

    x = x_ref[...]
    h = jnp.maximum(
        jnp.dot(x, w1_ref[...] * s1, preferred_element_type=jnp.float32)
        + c1, 0.0)
    h = jnp.maximum(
        jnp.dot(h, w2_ref[...] * s2, preferred_element_type=jnp.float32)
        + c2, 0.0)
    logits = (jnp.dot(h, w3_ref[...], preferred_element_type=jnp.float32)
              + b3_ref[...])

    m = jnp.max(logits, axis=1, keepdims=True)
    lse = m + jnp.log(jnp.sum(jnp.exp(logits - m), axis=1, keepdims=True))

    lane = jax.lax.broadcasted_iota(jnp.int32, (_BM, _L), 1)
    onehot = lane == t_ref[...]
    part = (jnp.sum(jnp.where(onehot, logits, 0.0)) - jnp.sum(lse))

    @pl.when(i == 0)
    def _():
        out_ref[0, 0] = 0.0

    out_ref[0, 0] += part

    @pl.when(i == _GRID - 1)
    def _():
        out_ref[0, 0] = out_ref[0, 0] * (-1.0 / _B)


def kernel(x_start, target, W1, b1, g1, beta1, rm1, rv1, W2, b2, g2, beta2,
           rm2, rv2, W3, b3):
    tgt = target.astype(jnp.int32).reshape(_B, 1)
    row = lambda v: v.reshape(1, -1)

    full = lambda shape: pl.BlockSpec(shape, lambda i: tuple(0 for _ in shape))
    loss = pl.pallas_call(
        _fused_kernel,
        grid=(_GRID,),
        in_specs=[
            pl.BlockSpec((_BM, _D), lambda i: (i, 0)),
            pl.BlockSpec((_BM, 1), lambda i: (i, 0)),
            full((_D, _H)),
            full((1, _H)), full((1, _H)), full((1, _H)),
            full((1, _H)), full((1, _H)),
            full((_H, _H)),
            full((1, _H)), full((1, _H)), full((1, _H)),
            full((1, _H)), full((1, _H)),
            full((_H, _L)),
            full((1, _L)),
        ],
        out_specs=pl.BlockSpec(memory_space=pltpu.SMEM),
        out_shape=jax.ShapeDtypeStruct((1, 1), jnp.float32),
    )(x_start, tgt, W1, row(b1), row(g1), row(beta1), row(rm1), row(rv1),
      W2, row(b2), row(g2), row(beta2), row(rm2), row(rv2), W3, row(b3))
    return loss[0, 0]


# speedup vs baseline: 2.0811x; 1.2988x over previous
import jax
import jax.numpy as jnp
from jax.experimental import pallas as pl
from jax.experimental.pallas import tpu as pltpu

_B, _D, _H, _L = 16384, 32, 64, 100
_BM = 8192
_GRID = _B // _BM


def _fused_kernel(x_ref, t_ref, w1_ref, b1_ref, g1_ref, beta1_ref, rm1_ref,
                  rv1_ref, w2_ref, b2_ref, g2_ref, beta2_ref, rm2_ref,
                  rv2_ref, w3_ref, b3_ref, out_ref):
    i = pl.program_id(0)
    eps = 1e-5

    s1 = g1_ref[...] * jax.lax.rsqrt(rv1_ref[...] + eps)
    c1 = (b1_ref[...] - rm1_ref[...]) * s1 + beta1_ref[...]
    s2 = g2_ref[...] * jax.lax.rsqrt(rv2_ref[...] + eps)
    c2 = (b2_ref[...] - rm2_ref[...]) * s2 + beta2_ref[...]

    x = x_ref[...]
    h = jnp.maximum(
        jnp.dot(x, w1_ref[...] * s1, preferred_element_type=jnp.float32)
        + c1, 0.0)
    h = jnp.maximum(
        jnp.dot(h, w2_ref[...] * s2, preferred_element_type=jnp.float32)
        + c2, 0.0)
    logits = (jnp.dot(h, w3_ref[...], preferred_element_type=jnp.float32)
              + b3_ref[...])

    m = jnp.max(logits, axis=1, keepdims=True)
    lse = m + jnp.log(jnp.sum(jnp.exp(logits - m), axis=1, keepdims=True))

    lane = jax.lax.broadcasted_iota(jnp.int32, (_BM, _L), 1)
    onehot = lane == t_ref[...].reshape(_BM, 1)
    part = (jnp.sum(jnp.where(onehot, logits, 0.0)) - jnp.sum(lse))

    @pl.when(i == 0)
    def _():
        out_ref[0, 0] = 0.0

    out_ref[0, 0] += part

    @pl.when(i == _GRID - 1)
    def _():
        out_ref[0, 0] = out_ref[0, 0] * (-1.0 / _B)


def kernel(x_start, target, W1, b1, g1, beta1, rm1, rv1, W2, b2, g2, beta2,
           rm2, rv2, W3, b3):
    tgt = target.astype(jnp.int32)
    row = lambda v: v.reshape(1, -1)

    full = lambda shape: pl.BlockSpec(shape, lambda i: tuple(0 for _ in shape))
    loss = pl.pallas_call(
        _fused_kernel,
        grid=(_GRID,),
        in_specs=[
            pl.BlockSpec((_BM, _D), lambda i: (i, 0)),
            pl.BlockSpec((_BM,), lambda i: (i,)),
            full((_D, _H)),
            full((1, _H)), full((1, _H)), full((1, _H)),
            full((1, _H)), full((1, _H)),
            full((_H, _H)),
            full((1, _H)), full((1, _H)), full((1, _H)),
            full((1, _H)), full((1, _H)),
            full((_H, _L)),
            full((1, _L)),
        ],
        out_specs=pl.BlockSpec(memory_space=pltpu.SMEM),
        out_shape=jax.ShapeDtypeStruct((1, 1), jnp.float32),
    )(x_start, tgt, W1, row(b1), row(g1), row(beta1), row(rm1), row(rv1),
      W2, row(b2), row(g2), row(beta2), row(rm2), row(rv2), W3, row(b3))
    return loss[0, 0]
